# baseline (speedup 1.0000x reference)
import functools

import jax
import jax.numpy as jnp
from jax import lax
from jax.experimental import pallas as pl
from jax.experimental.pallas import tpu as pltpu
from jax.experimental.pallas import tpu_sc as plsc

NC = 2
NS = 16
CHUNK = 64
AUG = 16
IDX_BLK = 40


def _sc_aggregate(x_aug, gidx, sidx, zeros_init, n_rows, n_chunks, da):
  rows_per_tile = n_rows // NS
  mesh = plsc.VectorSubcoreMesh(
      core_axis_name="c", subcore_axis_name="s", num_cores=NC, num_subcores=NS)

  @functools.partial(
      pl.kernel,
      out_type=jax.ShapeDtypeStruct((NC, n_rows, da), jnp.float32),
      mesh=mesh,
      compiler_params=pltpu.CompilerParams(use_tc_tiling_on_sc=False),
      scratch_types=[
          pltpu.VMEM((2 * IDX_BLK, CHUNK), jnp.int32),
          pltpu.VMEM((2 * IDX_BLK, CHUNK), jnp.int32),
          pltpu.VMEM((2, CHUNK, da), jnp.float32),
          pltpu.VMEM_SHARED((n_rows, da), jnp.float32),
          pltpu.SemaphoreType.DMA,
          pltpu.SemaphoreType.DMA,
          pltpu.SemaphoreType.DMA,
      ],
  )
  def agg(x_hbm, g_hbm, s_hbm, z_hbm, out_hbm, g_v, s_v, rows_v, acc,
          gsem, ssem, isem):
    c = lax.axis_index("c")
    s = lax.axis_index("s")
    r0 = s * rows_per_tile
    n_blocks = n_chunks // IDX_BLK
    pltpu.sync_copy(z_hbm, acc.at[pl.ds(r0, rows_per_tile), :])

    def stage_idx(bi):
      slot = (bi % 2) * IDX_BLK
      src = pl.ds(bi * IDX_BLK, IDX_BLK)
      dstv = pl.ds(slot, IDX_BLK)
      pltpu.async_copy(g_hbm.at[c, s, src], g_v.at[dstv], isem)
      pltpu.async_copy(s_hbm.at[c, s, src], s_v.at[dstv], isem)

    def wait_idx():
      pltpu.make_async_copy(g_hbm.at[0, 0, pl.ds(0, IDX_BLK)],
                            g_v.at[pl.ds(0, IDX_BLK)], isem).wait()
      pltpu.make_async_copy(s_hbm.at[0, 0, pl.ds(0, IDX_BLK)],
                            s_v.at[pl.ds(0, IDX_BLK)], isem).wait()

    def start_gather(j, p):
      pltpu.async_copy(x_hbm.at[g_v.at[j]], rows_v.at[p], gsem)

    def wait_gather():
      pltpu.make_async_copy(x_hbm.at[g_v.at[0]], rows_v.at[0], gsem).wait()

    def start_scatter(j, p):
      pltpu.async_copy(rows_v.at[p], acc.at[s_v.at[j]], ssem, add=True)

    def wait_scatter():
      pltpu.make_async_copy(rows_v.at[0], acc.at[s_v.at[0]], ssem).wait()

    stage_idx(0)
    wait_idx()
    plsc.subcore_barrier()
    start_gather(0, 0)

    def body(j, carry):
      jmod = j % (2 * IDX_BLK)
      p = j % 2

      wait_gather()

      @pl.when(j > 0)
      def _():
        wait_scatter()

      @pl.when(jnp.logical_and(j % IDX_BLK == 0, j // IDX_BLK + 1 < n_blocks))
      def _():
        stage_idx(j // IDX_BLK + 1)

      @pl.when(j + 1 < n_chunks)
      def _():
        @pl.when((j + 1) % IDX_BLK == 0)
        def _():
          wait_idx()
        start_gather((j + 1) % (2 * IDX_BLK), 1 - p)

      start_scatter(jmod, p)
      return carry

    lax.fori_loop(0, n_chunks, body, 0)
    wait_scatter()
    plsc.subcore_barrier()
    pltpu.sync_copy(acc.at[pl.ds(r0, rows_per_tile), :],
                    out_hbm.at[c, pl.ds(r0, rows_per_tile), :])

  return agg(x_aug, gidx, sidx, zeros_init)


def _tc_combine(x, acc, wt, b2, d_in, d_out, da):
  n = x.shape[0]
  blk = 1000
  grid = (n // blk,)

  def body(x_ref, ai_ref, ao_ref, w_ref, b_ref, o_ref):
    xb = x_ref[...]
    ai = ai_ref[0]
    ao = ao_ref[0]
    mi = ai[:, :d_in] / jnp.maximum(ai[:, d_in:d_in + 1], 1.0)
    mo = ao[:, :d_in] / jnp.maximum(ao[:, d_in:d_in + 1], 1.0)
    w = w_ref[...]
    o = (jnp.dot(xb, w[:d_in], preferred_element_type=jnp.float32)
         + jnp.dot(mi, w[d_in:2 * d_in], preferred_element_type=jnp.float32)
         + jnp.dot(mo, w[2 * d_in:3 * d_in], preferred_element_type=jnp.float32))
    o_ref[...] = jnp.maximum(o + b_ref[...], 0.0)

  return pl.pallas_call(
      body,
      grid=grid,
      in_specs=[
          pl.BlockSpec((blk, d_in), lambda i: (i, 0)),
          pl.BlockSpec((1, blk, da), lambda i: (0, i, 0)),
          pl.BlockSpec((1, blk, da), lambda i: (1, i, 0)),
          pl.BlockSpec((3 * d_in, d_out), lambda i: (0, 0)),
          pl.BlockSpec((1, d_out), lambda i: (0, 0)),
      ],
      out_specs=pl.BlockSpec((blk, d_out), lambda i: (i, 0)),
      out_shape=jax.ShapeDtypeStruct((n, d_out), jnp.float32),
  )(x, acc, acc, wt, b2)


def kernel(x, edge_index, W, b):
  n, d_in = x.shape
  d_out = W.shape[0]
  da = d_in + AUG
  e = edge_index.shape[1]

  src = edge_index[0].astype(jnp.int32)
  dst = edge_index[1].astype(jnp.int32)

  chunks = -(-e // (NS * CHUNK))
  per_tile_chunks = -(-chunks // IDX_BLK) * IDX_BLK
  e_pad = per_tile_chunks * NS * CHUNK
  pad = e_pad - e
  g0 = jnp.pad(src, (0, pad))
  g1 = jnp.pad(dst, (0, pad))
  s0 = jnp.pad(dst, (0, pad), constant_values=n)
  s1 = jnp.pad(src, (0, pad), constant_values=n)
  gidx = jnp.stack([g0, g1]).reshape(NC, NS, per_tile_chunks, CHUNK)
  sidx = jnp.stack([s0, s1]).reshape(NC, NS, per_tile_chunks, CHUNK)

  n_rows = -(-(n + 1) // (NS * 8)) * NS * 8

  x_aug = jnp.concatenate(
      [x, jnp.ones((n, 1), jnp.float32), jnp.zeros((n, AUG - 1), jnp.float32)],
      axis=1)
  zeros_init = jnp.zeros((n_rows // NS, da), jnp.float32)

  acc = _sc_aggregate(x_aug, gidx, sidx, zeros_init, n_rows, per_tile_chunks, da)

  wt = W.T
  b2 = b.reshape(1, d_out)
  return _tc_combine(x, acc, wt, b2, d_in, d_out, da)


# speedup vs baseline: 1.1215x; 1.1215x over previous
import functools

import jax
import jax.numpy as jnp
from jax import lax
from jax.experimental import pallas as pl
from jax.experimental.pallas import tpu as pltpu
from jax.experimental.pallas import tpu_sc as plsc

NC = 2
NS = 16
CHUNK = 128
LANES = 16
IDX_BLK = 16


def _sc_sums(x, gidx, sidx, zeros_init, n_rows, n_chunks, d):
  rows_per_tile = n_rows // NS
  mesh = plsc.VectorSubcoreMesh(
      core_axis_name="c", subcore_axis_name="s", num_cores=NC, num_subcores=NS)

  @functools.partial(
      pl.kernel,
      out_type=jax.ShapeDtypeStruct((NC, n_rows, d), jnp.float32),
      mesh=mesh,
      scratch_types=[
          pltpu.VMEM((2 * IDX_BLK, CHUNK), jnp.int32),
          pltpu.VMEM((2 * IDX_BLK, CHUNK), jnp.int32),
          pltpu.VMEM((2, CHUNK, d), jnp.float32),
          pltpu.VMEM_SHARED((n_rows, d), jnp.float32),
          pltpu.SemaphoreType.DMA,
          pltpu.SemaphoreType.DMA,
          pltpu.SemaphoreType.DMA,
      ],
  )
  def agg(x_hbm, g_hbm, s_hbm, z_hbm, out_hbm, g_v, s_v, rows_v, acc,
          gsem, ssem, isem):
    c = lax.axis_index("c")
    s = lax.axis_index("s")
    r0 = s * rows_per_tile
    n_blocks = n_chunks // IDX_BLK
    pltpu.sync_copy(z_hbm, acc.at[pl.ds(r0, rows_per_tile), :])

    def stage_idx(bi):
      slot = (bi % 2) * IDX_BLK
      src = pl.ds(bi * IDX_BLK, IDX_BLK)
      dstv = pl.ds(slot, IDX_BLK)
      pltpu.async_copy(g_hbm.at[c, s, src], g_v.at[dstv], isem)
      pltpu.async_copy(s_hbm.at[c, s, src], s_v.at[dstv], isem)

    def wait_idx():
      pltpu.make_async_copy(g_hbm.at[0, 0, pl.ds(0, IDX_BLK)],
                            g_v.at[pl.ds(0, IDX_BLK)], isem).wait()
      pltpu.make_async_copy(s_hbm.at[0, 0, pl.ds(0, IDX_BLK)],
                            s_v.at[pl.ds(0, IDX_BLK)], isem).wait()

    def start_gather(j, p):
      pltpu.async_copy(x_hbm.at[g_v.at[j]], rows_v.at[p], gsem)

    def wait_gather():
      pltpu.make_async_copy(x_hbm.at[g_v.at[0]], rows_v.at[0], gsem).wait()

    def start_scatter(j, p):
      pltpu.async_copy(rows_v.at[p], acc.at[s_v.at[j]], ssem, add=True)

    def wait_scatter():
      pltpu.make_async_copy(rows_v.at[0], acc.at[s_v.at[0]], ssem).wait()

    stage_idx(0)
    wait_idx()
    plsc.subcore_barrier()
    start_gather(0, 0)

    def body(j, carry):
      p = j % 2
      wait_gather()

      @pl.when(j > 0)
      def _():
        wait_scatter()

      @pl.when(jnp.logical_and(j % IDX_BLK == 0, j // IDX_BLK + 1 < n_blocks))
      def _():
        stage_idx(j // IDX_BLK + 1)

      @pl.when(j + 1 < n_chunks)
      def _():
        @pl.when((j + 1) % IDX_BLK == 0)
        def _():
          wait_idx()
        start_gather((j + 1) % (2 * IDX_BLK), 1 - p)

      start_scatter(j % (2 * IDX_BLK), p)
      return carry

    lax.fori_loop(0, n_chunks, body, 0)
    wait_scatter()
    plsc.subcore_barrier()
    pltpu.sync_copy(acc.at[pl.ds(r0, rows_per_tile), :],
                    out_hbm.at[c, pl.ds(r0, rows_per_tile), :])

  return agg(x, gidx, sidx, zeros_init)


def _sc_degrees(sidx, zeros_init, n_rows, n_chunks):
  deg_rows = n_rows // 128
  mesh = plsc.VectorSubcoreMesh(
      core_axis_name="c", subcore_axis_name="s", num_cores=NC, num_subcores=NS)

  @functools.partial(
      pl.kernel,
      out_type=jax.ShapeDtypeStruct((NC, deg_rows, 128), jnp.float32),
      mesh=mesh,
      compiler_params=pltpu.CompilerParams(use_tc_tiling_on_sc=False,
                                           needs_layout_passes=False),
      scratch_types=[
          pltpu.VMEM((n_chunks, CHUNK), jnp.int32),
          pltpu.VMEM((deg_rows, 128), jnp.float32),
          pltpu.VMEM((deg_rows,), jnp.int32),
          pltpu.VMEM_SHARED((deg_rows, 128), jnp.float32),
      ],
  )
  def deg(s_hbm, z_hbm, deg_hbm, s_v, deg_h, deg_i, deg_acc):
    c = lax.axis_index("c")
    s = lax.axis_index("s")
    pltpu.sync_copy(s_hbm.at[c, s], s_v)
    pltpu.sync_copy(z_hbm, deg_h)

    @pl.when(s == 0)
    def _():
      pltpu.sync_copy(z_hbm, deg_acc)

    for k in range(deg_rows // LANES):
      deg_i[pl.ds(k * LANES, LANES)] = lax.iota(jnp.int32, LANES) + k * LANES
    plsc.subcore_barrier()
    ones = jnp.ones((LANES,), jnp.float32)

    def body(j, carry):
      for k in range(CHUNK // LANES):
        v = s_v[j, pl.ds(k * LANES, LANES)]
        hi = lax.shift_right_logical(v, 7)
        lo = lax.bitwise_and(v, 127)
        plsc.addupdate_scatter(deg_h, [hi, lo], ones)
      return carry

    lax.fori_loop(0, n_chunks, body, 0)
    pltpu.sync_copy(deg_h, deg_acc.at[deg_i], add=True)
    plsc.subcore_barrier()

    @pl.when(s == 0)
    def _():
      pltpu.sync_copy(deg_acc, deg_hbm.at[c])

  return deg(sidx, zeros_init)


def _tc_combine(x, acc, deg, wt, b2, d_in, d_out):
  n = x.shape[0]
  blk = 1000
  grid = (n // blk,)

  def body(x_ref, ai_ref, ao_ref, di_ref, do_ref, w_ref, b_ref, o_ref):
    xb = x_ref[...]
    mi = ai_ref[0] / jnp.maximum(di_ref[...], 1.0)
    mo = ao_ref[0] / jnp.maximum(do_ref[...], 1.0)
    w = w_ref[...]
    o = (jnp.dot(xb, w[:d_in], preferred_element_type=jnp.float32)
         + jnp.dot(mi, w[d_in:2 * d_in], preferred_element_type=jnp.float32)
         + jnp.dot(mo, w[2 * d_in:3 * d_in], preferred_element_type=jnp.float32))
    o_ref[...] = jnp.maximum(o + b_ref[...], 0.0)

  nb = n // blk
  return pl.pallas_call(
      body,
      grid=grid,
      in_specs=[
          pl.BlockSpec((blk, d_in), lambda i: (i, 0)),
          pl.BlockSpec((1, blk, d_in), lambda i: (0, i, 0)),
          pl.BlockSpec((1, blk, d_in), lambda i: (1, i, 0)),
          pl.BlockSpec((blk, 1), lambda i: (i, 0)),
          pl.BlockSpec((blk, 1), lambda i: (nb + i, 0)),
          pl.BlockSpec((3 * d_in, d_out), lambda i: (0, 0)),
          pl.BlockSpec((1, d_out), lambda i: (0, 0)),
      ],
      out_specs=pl.BlockSpec((blk, d_out), lambda i: (i, 0)),
      out_shape=jax.ShapeDtypeStruct((n, d_out), jnp.float32),
  )(x, acc, acc, deg, deg, wt, b2)


def kernel(x, edge_index, W, b):
  n, d_in = x.shape
  d_out = W.shape[0]
  e = edge_index.shape[1]

  src = edge_index[0].astype(jnp.int32)
  dst = edge_index[1].astype(jnp.int32)

  chunks = -(-e // (NS * CHUNK))
  per_tile_chunks = -(-chunks // IDX_BLK) * IDX_BLK
  e_pad = per_tile_chunks * NS * CHUNK
  pad = e_pad - e
  g0 = jnp.pad(src, (0, pad))
  g1 = jnp.pad(dst, (0, pad))
  s0 = jnp.pad(dst, (0, pad), constant_values=n)
  s1 = jnp.pad(src, (0, pad), constant_values=n)
  gidx = jnp.stack([g0, g1]).reshape(NC, NS, per_tile_chunks, CHUNK)
  sidx = jnp.stack([s0, s1]).reshape(NC, NS, per_tile_chunks, CHUNK)

  n_rows = -(-(n + 1) // (NS * 128)) * NS * 128
  deg_rows = n_rows // 128

  zeros_sums = jnp.zeros((n_rows // NS, d_in), jnp.float32)
  zeros_deg = jnp.zeros((deg_rows, 128), jnp.float32)

  acc = _sc_sums(x, gidx, sidx, zeros_sums, n_rows, per_tile_chunks, d_in)
  deg = _sc_degrees(sidx, zeros_deg, n_rows, per_tile_chunks)
  deg2 = deg.reshape(NC, n_rows)[:, :n].reshape(NC * n, 1)

  wt = W.T
  b2 = b.reshape(1, d_out)
  return _tc_combine(x, acc, deg2, wt, b2, d_in, d_out)


# speedup vs baseline: 1.7401x; 1.5516x over previous
import functools

import jax
import jax.numpy as jnp
from jax import lax
from jax.experimental import pallas as pl
from jax.experimental.pallas import tpu as pltpu
from jax.experimental.pallas import tpu_sc as plsc

NC = 2
NS = 16
CHUNK = 128
LANES = 16
IDX_BLK = 16


def _sc_sums(x, gidx, sidx, zeros_init, n_rows, n_chunks, d):
  rows_per_tile = n_rows // NS
  mesh = plsc.VectorSubcoreMesh(
      core_axis_name="c", subcore_axis_name="s", num_cores=NC, num_subcores=NS)

  @functools.partial(
      pl.kernel,
      out_type=jax.ShapeDtypeStruct((NC, 2, n_rows, d), jnp.bfloat16),
      mesh=mesh,
      compiler_params=pltpu.CompilerParams(use_tc_tiling_on_sc=False),
      scratch_types=[
          pltpu.VMEM((2 * IDX_BLK, CHUNK), jnp.int32),
          pltpu.VMEM((2 * IDX_BLK, CHUNK), jnp.int32),
          pltpu.VMEM((2, CHUNK, d), jnp.bfloat16),
          pltpu.VMEM_SHARED((2, n_rows, d), jnp.bfloat16),
          pltpu.SemaphoreType.DMA,
          pltpu.SemaphoreType.DMA,
          pltpu.SemaphoreType.DMA,
      ],
  )
  def agg(x_hbm, g_hbm, s_hbm, z_hbm, out_hbm, g_v, s_v, rows_v, acc2,
          gsem, ssem, isem):
    c = lax.axis_index("c")
    s = lax.axis_index("s")
    r0 = s * rows_per_tile
    h = s % 2
    acc = acc2.at[h]
    n_blocks = n_chunks // IDX_BLK
    pltpu.sync_copy(z_hbm, acc2.at[0, pl.ds(r0, rows_per_tile), :])
    pltpu.sync_copy(z_hbm, acc2.at[1, pl.ds(r0, rows_per_tile), :])

    def stage_idx(bi):
      slot = (bi % 2) * IDX_BLK
      src = pl.ds(bi * IDX_BLK, IDX_BLK)
      dstv = pl.ds(slot, IDX_BLK)
      pltpu.async_copy(g_hbm.at[c, s, src], g_v.at[dstv], isem)
      pltpu.async_copy(s_hbm.at[c, s, src], s_v.at[dstv], isem)

    def wait_idx():
      pltpu.make_async_copy(g_hbm.at[0, 0, pl.ds(0, IDX_BLK)],
                            g_v.at[pl.ds(0, IDX_BLK)], isem).wait()
      pltpu.make_async_copy(s_hbm.at[0, 0, pl.ds(0, IDX_BLK)],
                            s_v.at[pl.ds(0, IDX_BLK)], isem).wait()

    def start_gather(j, p):
      pltpu.async_copy(x_hbm.at[g_v.at[j]], rows_v.at[p], gsem)

    def wait_gather():
      pltpu.make_async_copy(x_hbm.at[g_v.at[0]], rows_v.at[0], gsem).wait()

    def start_scatter(j, p):
      pltpu.async_copy(rows_v.at[p], acc.at[s_v.at[j]], ssem, add=True)

    def wait_scatter():
      pltpu.make_async_copy(rows_v.at[0], acc.at[s_v.at[0]], ssem).wait()

    stage_idx(0)
    wait_idx()
    plsc.subcore_barrier()
    start_gather(0, 0)

    def body(j, carry):
      p = j % 2
      wait_gather()

      @pl.when(j > 0)
      def _():
        wait_scatter()

      @pl.when(jnp.logical_and(j % IDX_BLK == 0, j // IDX_BLK + 1 < n_blocks))
      def _():
        stage_idx(j // IDX_BLK + 1)

      @pl.when(j + 1 < n_chunks)
      def _():
        @pl.when((j + 1) % IDX_BLK == 0)
        def _():
          wait_idx()
        start_gather((j + 1) % (2 * IDX_BLK), 1 - p)

      start_scatter(j % (2 * IDX_BLK), p)
      return carry

    lax.fori_loop(0, n_chunks, body, 0)
    wait_scatter()
    plsc.subcore_barrier()
    pltpu.sync_copy(acc2.at[0, pl.ds(r0, rows_per_tile), :],
                    out_hbm.at[c, 0, pl.ds(r0, rows_per_tile), :])
    pltpu.sync_copy(acc2.at[1, pl.ds(r0, rows_per_tile), :],
                    out_hbm.at[c, 1, pl.ds(r0, rows_per_tile), :])

  return agg(x, gidx, sidx, zeros_init)


def _sc_degrees(sidx, zeros_init, n_rows, n_chunks):
  deg_rows = n_rows // 128
  mesh = plsc.VectorSubcoreMesh(
      core_axis_name="c", subcore_axis_name="s", num_cores=NC, num_subcores=NS)

  @functools.partial(
      pl.kernel,
      out_type=jax.ShapeDtypeStruct((NC, deg_rows, 128), jnp.float32),
      mesh=mesh,
      compiler_params=pltpu.CompilerParams(use_tc_tiling_on_sc=False,
                                           needs_layout_passes=False),
      scratch_types=[
          pltpu.VMEM((n_chunks, CHUNK), jnp.int32),
          pltpu.VMEM((deg_rows, 128), jnp.float32),
          pltpu.VMEM((deg_rows,), jnp.int32),
          pltpu.VMEM_SHARED((deg_rows, 128), jnp.float32),
      ],
  )
  def deg(s_hbm, z_hbm, deg_hbm, s_v, deg_h, deg_i, deg_acc):
    c = lax.axis_index("c")
    s = lax.axis_index("s")
    pltpu.sync_copy(s_hbm.at[c, s], s_v)
    pltpu.sync_copy(z_hbm, deg_h)

    @pl.when(s == 0)
    def _():
      pltpu.sync_copy(z_hbm, deg_acc)

    for k in range(deg_rows // LANES):
      deg_i[pl.ds(k * LANES, LANES)] = lax.iota(jnp.int32, LANES) + k * LANES
    plsc.subcore_barrier()
    ones = jnp.ones((LANES,), jnp.float32)

    def body(j, carry):
      for k in range(CHUNK // LANES):
        v = s_v[j, pl.ds(k * LANES, LANES)]
        hi = lax.shift_right_logical(v, 7)
        lo = lax.bitwise_and(v, 127)
        plsc.addupdate_scatter(deg_h, [hi, lo], ones)
      return carry

    lax.fori_loop(0, n_chunks, body, 0)
    pltpu.sync_copy(deg_h, deg_acc.at[deg_i], add=True)
    plsc.subcore_barrier()

    @pl.when(s == 0)
    def _():
      pltpu.sync_copy(deg_acc, deg_hbm.at[c])

  return deg(sidx, zeros_init)


def _tc_combine(x, acc, deg, wt, b2, d_in, d_out):
  n = x.shape[0]
  blk = 1000
  grid = (n // blk,)

  def body(x_ref, ai_ref, ao_ref, di_ref, do_ref, w_ref, b_ref, o_ref):
    xb = x_ref[...]
    si = ai_ref[0, 0].astype(jnp.float32) + ai_ref[0, 1].astype(jnp.float32)
    so = ao_ref[0, 0].astype(jnp.float32) + ao_ref[0, 1].astype(jnp.float32)
    mi = si / jnp.maximum(di_ref[...], 1.0)
    mo = so / jnp.maximum(do_ref[...], 1.0)
    w = w_ref[...]
    o = (jnp.dot(xb, w[:d_in], preferred_element_type=jnp.float32)
         + jnp.dot(mi, w[d_in:2 * d_in], preferred_element_type=jnp.float32)
         + jnp.dot(mo, w[2 * d_in:3 * d_in], preferred_element_type=jnp.float32))
    o_ref[...] = jnp.maximum(o + b_ref[...], 0.0)

  nb = n // blk
  return pl.pallas_call(
      body,
      grid=grid,
      in_specs=[
          pl.BlockSpec((blk, d_in), lambda i: (i, 0)),
          pl.BlockSpec((1, 2, blk, d_in), lambda i: (0, 0, i, 0)),
          pl.BlockSpec((1, 2, blk, d_in), lambda i: (1, 0, i, 0)),
          pl.BlockSpec((blk, 1), lambda i: (i, 0)),
          pl.BlockSpec((blk, 1), lambda i: (nb + i, 0)),
          pl.BlockSpec((3 * d_in, d_out), lambda i: (0, 0)),
          pl.BlockSpec((1, d_out), lambda i: (0, 0)),
      ],
      out_specs=pl.BlockSpec((blk, d_out), lambda i: (i, 0)),
      out_shape=jax.ShapeDtypeStruct((n, d_out), jnp.float32),
  )(x, acc, acc, deg, deg, wt, b2)


def kernel(x, edge_index, W, b):
  n, d_in = x.shape
  d_out = W.shape[0]
  e = edge_index.shape[1]

  src = edge_index[0].astype(jnp.int32)
  dst = edge_index[1].astype(jnp.int32)

  chunks = -(-e // (NS * CHUNK))
  per_tile_chunks = -(-chunks // IDX_BLK) * IDX_BLK
  e_pad = per_tile_chunks * NS * CHUNK
  pad = e_pad - e
  g0 = jnp.pad(src, (0, pad))
  g1 = jnp.pad(dst, (0, pad))
  s0 = jnp.pad(dst, (0, pad), constant_values=n)
  s1 = jnp.pad(src, (0, pad), constant_values=n)
  gidx = jnp.stack([g0, g1]).reshape(NC, NS, per_tile_chunks, CHUNK)
  sidx = jnp.stack([s0, s1]).reshape(NC, NS, per_tile_chunks, CHUNK)

  n_rows = -(-(n + 1) // (NS * 128)) * NS * 128
  deg_rows = n_rows // 128

  zeros_sums = jnp.zeros((n_rows // NS, d_in), jnp.bfloat16)
  zeros_deg = jnp.zeros((deg_rows, 128), jnp.float32)

  x_bf = x.astype(jnp.bfloat16)
  acc = _sc_sums(x_bf, gidx, sidx, zeros_sums, n_rows, per_tile_chunks, d_in)
  deg = _sc_degrees(sidx, zeros_deg, n_rows, per_tile_chunks)
  deg2 = deg.reshape(NC, n_rows)[:, :n].reshape(NC * n, 1)

  wt = W.T
  b2 = b.reshape(1, d_out)
  return _tc_combine(x, acc, deg2, wt, b2, d_in, d_out)


# speedup vs baseline: 3.7896x; 2.1778x over previous
import functools

import jax
import jax.numpy as jnp
from jax import lax
from jax.experimental import pallas as pl
from jax.experimental.pallas import tpu as pltpu
from jax.experimental.pallas import tpu_sc as plsc

NC = 2
NS = 16
CHUNK = 128
LANES = 16
IDX_BLK = 16


def _sc_sums(x, gidx, sidx, zeros_init, n_rows, n_chunks, d):
  rows_per_tile = n_rows // NS
  mesh = plsc.VectorSubcoreMesh(
      core_axis_name="c", subcore_axis_name="s", num_cores=NC, num_subcores=NS)

  @functools.partial(
      pl.kernel,
      out_type=jax.ShapeDtypeStruct((NC, n_rows, d), jnp.bfloat16),
      mesh=mesh,
      compiler_params=pltpu.CompilerParams(use_tc_tiling_on_sc=False),
      scratch_types=[
          pltpu.VMEM((2 * IDX_BLK, CHUNK), jnp.int32),
          pltpu.VMEM((2 * IDX_BLK, CHUNK), jnp.int32),
          pltpu.VMEM((2, CHUNK, d), jnp.bfloat16),
          pltpu.VMEM_SHARED((x.shape[0], d), jnp.bfloat16),
          pltpu.VMEM_SHARED((n_rows, d), jnp.bfloat16),
          pltpu.SemaphoreType.DMA,
          pltpu.SemaphoreType.DMA,
          pltpu.SemaphoreType.DMA,
      ],
  )
  def agg(x_hbm, g_hbm, s_hbm, z_hbm, out_hbm, g_v, s_v, rows_v, x_sh, acc,
          gsem, ssem, isem):
    c = lax.axis_index("c")
    s = lax.axis_index("s")
    r0 = s * rows_per_tile
    n_blocks = n_chunks // IDX_BLK
    n_x = x_hbm.shape[0]
    xs0 = s * (n_x // NS)
    pltpu.sync_copy(x_hbm.at[pl.ds(xs0, n_x // NS)],
                    x_sh.at[pl.ds(xs0, n_x // NS)])
    pltpu.sync_copy(z_hbm, acc.at[pl.ds(r0, rows_per_tile), :])

    def stage_idx(bi):
      slot = (bi % 2) * IDX_BLK
      src = pl.ds(bi * IDX_BLK, IDX_BLK)
      dstv = pl.ds(slot, IDX_BLK)
      pltpu.async_copy(g_hbm.at[c, s, src], g_v.at[dstv], isem)
      pltpu.async_copy(s_hbm.at[c, s, src], s_v.at[dstv], isem)

    def wait_idx():
      pltpu.make_async_copy(g_hbm.at[0, 0, pl.ds(0, IDX_BLK)],
                            g_v.at[pl.ds(0, IDX_BLK)], isem).wait()
      pltpu.make_async_copy(s_hbm.at[0, 0, pl.ds(0, IDX_BLK)],
                            s_v.at[pl.ds(0, IDX_BLK)], isem).wait()

    def start_gather(j, p):
      pltpu.async_copy(x_sh.at[g_v.at[j]], rows_v.at[p], gsem)

    def wait_gather():
      pltpu.make_async_copy(x_sh.at[g_v.at[0]], rows_v.at[0], gsem).wait()

    def start_scatter(j, p):
      pltpu.async_copy(rows_v.at[p], acc.at[s_v.at[j]], ssem, add=True)

    def wait_scatter():
      pltpu.make_async_copy(rows_v.at[0], acc.at[s_v.at[0]], ssem).wait()

    stage_idx(0)
    wait_idx()
    plsc.subcore_barrier()
    start_gather(0, 0)

    def body(j, carry):
      p = j % 2
      wait_gather()

      @pl.when(j > 0)
      def _():
        wait_scatter()

      @pl.when(jnp.logical_and(j % IDX_BLK == 0, j // IDX_BLK + 1 < n_blocks))
      def _():
        stage_idx(j // IDX_BLK + 1)

      @pl.when(j + 1 < n_chunks)
      def _():
        @pl.when((j + 1) % IDX_BLK == 0)
        def _():
          wait_idx()
        start_gather((j + 1) % (2 * IDX_BLK), 1 - p)

      start_scatter(j % (2 * IDX_BLK), p)
      return carry

    lax.fori_loop(0, n_chunks, body, 0)
    wait_scatter()
    plsc.subcore_barrier()
    pltpu.sync_copy(acc.at[pl.ds(r0, rows_per_tile), :],
                    out_hbm.at[c, pl.ds(r0, rows_per_tile), :])

  return agg(x, gidx, sidx, zeros_init)


def _sc_degrees(sidx, zeros_init, n_rows, n_chunks):
  deg_rows = n_rows // 128
  mesh = plsc.VectorSubcoreMesh(
      core_axis_name="c", subcore_axis_name="s", num_cores=NC, num_subcores=NS)

  @functools.partial(
      pl.kernel,
      out_type=jax.ShapeDtypeStruct((NC, deg_rows, 128), jnp.float32),
      mesh=mesh,
      compiler_params=pltpu.CompilerParams(use_tc_tiling_on_sc=False,
                                           needs_layout_passes=False),
      scratch_types=[
          pltpu.VMEM((n_chunks, CHUNK), jnp.int32),
          pltpu.VMEM((deg_rows, 128), jnp.float32),
          pltpu.VMEM((deg_rows,), jnp.int32),
          pltpu.VMEM_SHARED((deg_rows, 128), jnp.float32),
      ],
  )
  def deg(s_hbm, z_hbm, deg_hbm, s_v, deg_h, deg_i, deg_acc):
    c = lax.axis_index("c")
    s = lax.axis_index("s")
    pltpu.sync_copy(s_hbm.at[c, s], s_v)
    pltpu.sync_copy(z_hbm, deg_h)

    @pl.when(s == 0)
    def _():
      pltpu.sync_copy(z_hbm, deg_acc)

    for k in range(deg_rows // LANES):
      deg_i[pl.ds(k * LANES, LANES)] = lax.iota(jnp.int32, LANES) + k * LANES
    plsc.subcore_barrier()
    ones = jnp.ones((LANES,), jnp.float32)

    def body(j, carry):
      for k in range(CHUNK // LANES):
        v = s_v[j, pl.ds(k * LANES, LANES)]
        hi = lax.shift_right_logical(v, 7)
        lo = lax.bitwise_and(v, 127)
        plsc.addupdate_scatter(deg_h, [hi, lo], ones)
      return carry

    lax.fori_loop(0, n_chunks, body, 0)
    pltpu.sync_copy(deg_h, deg_acc.at[deg_i], add=True)
    plsc.subcore_barrier()

    @pl.when(s == 0)
    def _():
      pltpu.sync_copy(deg_acc, deg_hbm.at[c])

  return deg(sidx, zeros_init)


def _tc_combine(x, acc, deg, wt, b2, d_in, d_out):
  n = x.shape[0]
  blk = 1000
  grid = (n // blk,)

  def body(x_ref, ai_ref, ao_ref, di_ref, do_ref, w_ref, b_ref, o_ref):
    xb = x_ref[...]
    mi = ai_ref[0].astype(jnp.float32) / jnp.maximum(di_ref[...], 1.0)
    mo = ao_ref[0].astype(jnp.float32) / jnp.maximum(do_ref[...], 1.0)
    w = w_ref[...]
    o = (jnp.dot(xb, w[:d_in], preferred_element_type=jnp.float32)
         + jnp.dot(mi, w[d_in:2 * d_in], preferred_element_type=jnp.float32)
         + jnp.dot(mo, w[2 * d_in:3 * d_in], preferred_element_type=jnp.float32))
    o_ref[...] = jnp.maximum(o + b_ref[...], 0.0)

  nb = n // blk
  return pl.pallas_call(
      body,
      grid=grid,
      in_specs=[
          pl.BlockSpec((blk, d_in), lambda i: (i, 0)),
          pl.BlockSpec((1, blk, d_in), lambda i: (0, i, 0)),
          pl.BlockSpec((1, blk, d_in), lambda i: (1, i, 0)),
          pl.BlockSpec((blk, 1), lambda i: (i, 0)),
          pl.BlockSpec((blk, 1), lambda i: (nb + i, 0)),
          pl.BlockSpec((3 * d_in, d_out), lambda i: (0, 0)),
          pl.BlockSpec((1, d_out), lambda i: (0, 0)),
      ],
      out_specs=pl.BlockSpec((blk, d_out), lambda i: (i, 0)),
      out_shape=jax.ShapeDtypeStruct((n, d_out), jnp.float32),
  )(x, acc, acc, deg, deg, wt, b2)


def kernel(x, edge_index, W, b):
  n, d_in = x.shape
  d_out = W.shape[0]
  e = edge_index.shape[1]

  src = edge_index[0].astype(jnp.int32)
  dst = edge_index[1].astype(jnp.int32)

  chunks = -(-e // (NS * CHUNK))
  per_tile_chunks = -(-chunks // IDX_BLK) * IDX_BLK
  e_pad = per_tile_chunks * NS * CHUNK
  pad = e_pad - e
  g0 = jnp.pad(src, (0, pad))
  g1 = jnp.pad(dst, (0, pad))
  s0 = jnp.pad(dst, (0, pad), constant_values=n)
  s1 = jnp.pad(src, (0, pad), constant_values=n)
  gidx = jnp.stack([g0, g1]).reshape(NC, NS, per_tile_chunks, CHUNK)
  sidx = jnp.stack([s0, s1]).reshape(NC, NS, per_tile_chunks, CHUNK)

  n_rows = -(-(n + 1) // (NS * 128)) * NS * 128
  deg_rows = n_rows // 128

  zeros_sums = jnp.zeros((n_rows // NS, d_in), jnp.bfloat16)
  zeros_deg = jnp.zeros((deg_rows, 128), jnp.float32)

  x_bf = x.astype(jnp.bfloat16)
  acc = _sc_sums(x_bf, gidx, sidx, zeros_sums, n_rows, per_tile_chunks, d_in)
  deg = _sc_degrees(sidx, zeros_deg, n_rows, per_tile_chunks)
  deg2 = deg.reshape(NC, n_rows)[:, :n].reshape(NC * n, 1)

  wt = W.T
  b2 = b.reshape(1, d_out)
  return _tc_combine(x, acc, deg2, wt, b2, d_in, d_out)


# speedup vs baseline: 3.9763x; 1.0493x over previous
import functools

import jax
import jax.numpy as jnp
from jax import lax
from jax.experimental import pallas as pl
from jax.experimental.pallas import tpu as pltpu
from jax.experimental.pallas import tpu_sc as plsc

NC = 2
NS = 16
CHUNK = 128
LANES = 16
IDX_BLK = 16


def _sc_sums(x, gidx, sidx, zeros_init, zeros_deg, n_rows, n_chunks, d):
  rows_per_tile = n_rows // NS
  deg_rows = n_rows // 128
  mesh = plsc.VectorSubcoreMesh(
      core_axis_name="c", subcore_axis_name="s", num_cores=NC, num_subcores=NS)

  @functools.partial(
      pl.kernel,
      out_type=(jax.ShapeDtypeStruct((NC, n_rows, d), jnp.bfloat16),
                jax.ShapeDtypeStruct((NC, deg_rows, 128), jnp.float32)),
      mesh=mesh,
      compiler_params=pltpu.CompilerParams(use_tc_tiling_on_sc=False,
                                           needs_layout_passes=False),
      scratch_types=[
          pltpu.VMEM((2 * IDX_BLK, CHUNK), jnp.int32),
          pltpu.VMEM((2 * IDX_BLK, CHUNK), jnp.int32),
          pltpu.VMEM((2, CHUNK, d), jnp.bfloat16),
          pltpu.VMEM((deg_rows, 128), jnp.float32),
          pltpu.VMEM((deg_rows,), jnp.int32),
          pltpu.VMEM_SHARED((x.shape[0], d), jnp.bfloat16),
          pltpu.VMEM_SHARED((n_rows, d), jnp.bfloat16),
          pltpu.VMEM_SHARED((deg_rows, 128), jnp.float32),
          pltpu.SemaphoreType.DMA,
          pltpu.SemaphoreType.DMA,
          pltpu.SemaphoreType.DMA,
      ],
  )
  def agg(x_hbm, g_hbm, s_hbm, z_hbm, zd_hbm, out_hbm, deg_hbm,
          g_v, s_v, rows_v, deg_h, deg_i, x_sh, acc, deg_acc,
          gsem, ssem, isem):
    c = lax.axis_index("c")
    s = lax.axis_index("s")
    r0 = s * rows_per_tile
    n_blocks = n_chunks // IDX_BLK
    n_x = x_hbm.shape[0]
    xs0 = s * (n_x // NS)
    pltpu.sync_copy(x_hbm.at[pl.ds(xs0, n_x // NS)],
                    x_sh.at[pl.ds(xs0, n_x // NS)])
    pltpu.sync_copy(z_hbm, acc.at[pl.ds(r0, rows_per_tile), :])
    pltpu.sync_copy(zd_hbm, deg_h)

    @pl.when(s == 0)
    def _():
      pltpu.sync_copy(zd_hbm, deg_acc)

    for k in range(deg_rows // LANES):
      deg_i[pl.ds(k * LANES, LANES)] = lax.iota(jnp.int32, LANES) + k * LANES

    def stage_idx(bi):
      slot = (bi % 2) * IDX_BLK
      src = pl.ds(bi * IDX_BLK, IDX_BLK)
      dstv = pl.ds(slot, IDX_BLK)
      pltpu.async_copy(g_hbm.at[c, s, src], g_v.at[dstv], isem)
      pltpu.async_copy(s_hbm.at[c, s, src], s_v.at[dstv], isem)

    def wait_idx():
      pltpu.make_async_copy(g_hbm.at[0, 0, pl.ds(0, IDX_BLK)],
                            g_v.at[pl.ds(0, IDX_BLK)], isem).wait()
      pltpu.make_async_copy(s_hbm.at[0, 0, pl.ds(0, IDX_BLK)],
                            s_v.at[pl.ds(0, IDX_BLK)], isem).wait()

    def start_gather(j, p):
      pltpu.async_copy(x_sh.at[g_v.at[j]], rows_v.at[p], gsem)

    def wait_gather():
      pltpu.make_async_copy(x_sh.at[g_v.at[0]], rows_v.at[0], gsem).wait()

    def start_scatter(j, p):
      pltpu.async_copy(rows_v.at[p], acc.at[s_v.at[j]], ssem, add=True)

    def wait_scatter():
      pltpu.make_async_copy(rows_v.at[0], acc.at[s_v.at[0]], ssem).wait()

    stage_idx(0)
    wait_idx()
    plsc.subcore_barrier()
    start_gather(0, 0)
    ones = jnp.ones((LANES,), jnp.float32)

    def body(j, carry):
      p = j % 2
      jmod = j % (2 * IDX_BLK)
      wait_gather()

      @pl.when(j > 0)
      def _():
        wait_scatter()

      @pl.when(jnp.logical_and(j % IDX_BLK == 0, j // IDX_BLK + 1 < n_blocks))
      def _():
        stage_idx(j // IDX_BLK + 1)

      @pl.when(j + 1 < n_chunks)
      def _():
        @pl.when((j + 1) % IDX_BLK == 0)
        def _():
          wait_idx()
        start_gather((j + 1) % (2 * IDX_BLK), 1 - p)

      start_scatter(jmod, p)
      for k in range(CHUNK // LANES):
        v = s_v[jmod, pl.ds(k * LANES, LANES)]
        hi = lax.shift_right_logical(v, 7)
        lo = lax.bitwise_and(v, 127)
        plsc.addupdate_scatter(deg_h, [hi, lo], ones)
      return carry

    lax.fori_loop(0, n_chunks, body, 0)
    wait_scatter()
    pltpu.sync_copy(deg_h, deg_acc.at[deg_i], add=True)
    plsc.subcore_barrier()
    pltpu.sync_copy(acc.at[pl.ds(r0, rows_per_tile), :],
                    out_hbm.at[c, pl.ds(r0, rows_per_tile), :])

    @pl.when(s == 0)
    def _():
      pltpu.sync_copy(deg_acc, deg_hbm.at[c])

  return agg(x, gidx, sidx, zeros_init, zeros_deg)


def _tc_combine(x, acc, deg, wt, b2, d_in, d_out):
  n = x.shape[0]
  blk = 1000
  grid = (n // blk,)

  def body(x_ref, ai_ref, ao_ref, di_ref, do_ref, w_ref, b_ref, o_ref):
    xb = x_ref[...]
    mi = ai_ref[0].astype(jnp.float32) / jnp.maximum(di_ref[...], 1.0)
    mo = ao_ref[0].astype(jnp.float32) / jnp.maximum(do_ref[...], 1.0)
    w = w_ref[...]
    o = (jnp.dot(xb, w[:d_in], preferred_element_type=jnp.float32)
         + jnp.dot(mi, w[d_in:2 * d_in], preferred_element_type=jnp.float32)
         + jnp.dot(mo, w[2 * d_in:3 * d_in], preferred_element_type=jnp.float32))
    o_ref[...] = jnp.maximum(o + b_ref[...], 0.0)

  nb = n // blk
  return pl.pallas_call(
      body,
      grid=grid,
      in_specs=[
          pl.BlockSpec((blk, d_in), lambda i: (i, 0)),
          pl.BlockSpec((1, blk, d_in), lambda i: (0, i, 0)),
          pl.BlockSpec((1, blk, d_in), lambda i: (1, i, 0)),
          pl.BlockSpec((blk, 1), lambda i: (i, 0)),
          pl.BlockSpec((blk, 1), lambda i: (nb + i, 0)),
          pl.BlockSpec((3 * d_in, d_out), lambda i: (0, 0)),
          pl.BlockSpec((1, d_out), lambda i: (0, 0)),
      ],
      out_specs=pl.BlockSpec((blk, d_out), lambda i: (i, 0)),
      out_shape=jax.ShapeDtypeStruct((n, d_out), jnp.float32),
  )(x, acc, acc, deg, deg, wt, b2)


def kernel(x, edge_index, W, b):
  n, d_in = x.shape
  d_out = W.shape[0]
  e = edge_index.shape[1]

  src = edge_index[0].astype(jnp.int32)
  dst = edge_index[1].astype(jnp.int32)

  chunks = -(-e // (NS * CHUNK))
  per_tile_chunks = -(-chunks // IDX_BLK) * IDX_BLK
  e_pad = per_tile_chunks * NS * CHUNK
  pad = e_pad - e
  g0 = jnp.pad(src, (0, pad))
  g1 = jnp.pad(dst, (0, pad))
  s0 = jnp.pad(dst, (0, pad), constant_values=n)
  s1 = jnp.pad(src, (0, pad), constant_values=n)
  gidx = jnp.stack([g0, g1]).reshape(NC, NS, per_tile_chunks, CHUNK)
  sidx = jnp.stack([s0, s1]).reshape(NC, NS, per_tile_chunks, CHUNK)

  n_rows = -(-(n + 1) // (NS * 128)) * NS * 128
  deg_rows = n_rows // 128

  zeros_sums = jnp.zeros((n_rows // NS, d_in), jnp.bfloat16)
  zeros_deg = jnp.zeros((deg_rows, 128), jnp.float32)

  x_bf = x.astype(jnp.bfloat16)
  acc, deg = _sc_sums(x_bf, gidx, sidx, zeros_sums, zeros_deg, n_rows,
                      per_tile_chunks, d_in)
  deg2 = deg.reshape(NC, n_rows)[:, :n].reshape(NC * n, 1)

  wt = W.T
  b2 = b.reshape(1, d_out)
  return _tc_combine(x, acc, deg2, wt, b2, d_in, d_out)


# speedup vs baseline: 4.0676x; 1.0230x over previous
import functools

import jax
import jax.numpy as jnp
from jax import lax
from jax.experimental import pallas as pl
from jax.experimental.pallas import tpu as pltpu
from jax.experimental.pallas import tpu_sc as plsc

NC = 2
NS = 16
CHUNK = 128
LANES = 16
IDX_BLK = 16


def _sc_sums(x, idx2, zeros_init, zeros_deg, n_rows, n_chunks, d):
  rows_per_tile = n_rows // NS
  deg_rows = n_rows // 128
  mesh = plsc.VectorSubcoreMesh(
      core_axis_name="c", subcore_axis_name="s", num_cores=NC, num_subcores=NS)

  @functools.partial(
      pl.kernel,
      out_type=(jax.ShapeDtypeStruct((NC, n_rows, d), jnp.bfloat16),
                jax.ShapeDtypeStruct((NC, deg_rows, 128), jnp.float32)),
      mesh=mesh,
      compiler_params=pltpu.CompilerParams(use_tc_tiling_on_sc=False,
                                           needs_layout_passes=False),
      scratch_types=[
          pltpu.VMEM((2 * IDX_BLK, CHUNK), jnp.int32),
          pltpu.VMEM((2 * IDX_BLK, CHUNK), jnp.int32),
          pltpu.VMEM((3, CHUNK, d), jnp.bfloat16),
          pltpu.VMEM((deg_rows, 128), jnp.float32),
          pltpu.VMEM((deg_rows,), jnp.int32),
          pltpu.VMEM_SHARED((x.shape[0] + 16, d), jnp.bfloat16),
          pltpu.VMEM_SHARED((n_rows, d), jnp.bfloat16),
          pltpu.VMEM_SHARED((deg_rows, 128), jnp.float32),
          pltpu.SemaphoreType.DMA,
          pltpu.SemaphoreType.DMA,
          pltpu.SemaphoreType.DMA,
      ],
  )
  def agg(x_hbm, idx_hbm, z_hbm, zd_hbm, out_hbm, deg_hbm,
          g_v, s_v, rows_v, deg_h, deg_i, x_sh, acc, deg_acc,
          gsem, ssem, isem):
    c = lax.axis_index("c")
    s = lax.axis_index("s")
    r0 = s * rows_per_tile
    n_blocks = n_chunks // IDX_BLK
    n_x = x_hbm.shape[0]
    xs0 = s * (n_x // NS)
    pltpu.sync_copy(x_hbm.at[pl.ds(xs0, n_x // NS)],
                    x_sh.at[pl.ds(xs0, n_x // NS)])
    pltpu.sync_copy(z_hbm, acc.at[pl.ds(r0, rows_per_tile), :])
    pltpu.sync_copy(zd_hbm, deg_h)

    @pl.when(s == 0)
    def _():
      pltpu.sync_copy(zd_hbm, deg_acc)

    for k in range(deg_rows // LANES):
      deg_i[pl.ds(k * LANES, LANES)] = lax.iota(jnp.int32, LANES) + k * LANES

    def stage_idx(bi):
      slot = (bi % 2) * IDX_BLK
      src = pl.ds(bi * IDX_BLK, IDX_BLK)
      dstv = pl.ds(slot, IDX_BLK)
      pltpu.async_copy(idx_hbm.at[c, s, src], g_v.at[dstv], isem)
      pltpu.async_copy(idx_hbm.at[1 - c, s, src], s_v.at[dstv], isem)

    def wait_idx():
      pltpu.make_async_copy(idx_hbm.at[0, 0, pl.ds(0, IDX_BLK)],
                            g_v.at[pl.ds(0, IDX_BLK)], isem).wait()
      pltpu.make_async_copy(idx_hbm.at[0, 0, pl.ds(0, IDX_BLK)],
                            s_v.at[pl.ds(0, IDX_BLK)], isem).wait()

    def start_gather(j):
      pltpu.async_copy(x_sh.at[g_v.at[j % (2 * IDX_BLK)]],
                       rows_v.at[j % 3], gsem)

    def wait_gather():
      pltpu.make_async_copy(x_sh.at[g_v.at[0]], rows_v.at[0], gsem).wait()

    def start_scatter(j):
      pltpu.async_copy(rows_v.at[j % 3], acc.at[s_v.at[j % (2 * IDX_BLK)]],
                       ssem, add=True)

    def wait_scatter():
      pltpu.make_async_copy(rows_v.at[0], acc.at[s_v.at[0]], ssem).wait()

    stage_idx(0)
    wait_idx()
    plsc.subcore_barrier()
    start_gather(0)
    start_gather(1)
    ones = jnp.ones((LANES,), jnp.float32)

    def body(j, carry):
      jmod = j % (2 * IDX_BLK)
      wait_gather()

      @pl.when(j > 0)
      def _():
        wait_scatter()

      @pl.when(jnp.logical_and(j % IDX_BLK == 0, j // IDX_BLK + 1 < n_blocks))
      def _():
        stage_idx(j // IDX_BLK + 1)

      @pl.when(j + 2 < n_chunks)
      def _():
        @pl.when((j + 2) % IDX_BLK == 0)
        def _():
          wait_idx()
        start_gather(j + 2)

      start_scatter(j)
      for k in range(CHUNK // LANES):
        v = s_v[jmod, pl.ds(k * LANES, LANES)]
        hi = lax.shift_right_logical(v, 7)
        lo = lax.bitwise_and(v, 127)
        plsc.addupdate_scatter(deg_h, [hi, lo], ones)
      return carry

    lax.fori_loop(0, n_chunks, body, 0)
    wait_scatter()
    pltpu.sync_copy(deg_h, deg_acc.at[deg_i], add=True)
    plsc.subcore_barrier()
    pltpu.sync_copy(acc.at[pl.ds(r0, rows_per_tile), :],
                    out_hbm.at[c, pl.ds(r0, rows_per_tile), :])

    @pl.when(s == 0)
    def _():
      pltpu.sync_copy(deg_acc, deg_hbm.at[c])

  return agg(x, idx2, zeros_init, zeros_deg)


def _tc_combine(x, acc, deg, wt, b2, d_in, d_out):
  n = x.shape[0]
  blk = 1000
  grid = (n // blk,)

  def body(x_ref, ai_ref, ao_ref, di_ref, do_ref, w_ref, b_ref, o_ref):
    xb = x_ref[...]
    mi = ai_ref[0].astype(jnp.float32) / jnp.maximum(di_ref[...], 1.0)
    mo = ao_ref[0].astype(jnp.float32) / jnp.maximum(do_ref[...], 1.0)
    w = w_ref[...]
    o = (jnp.dot(xb, w[:d_in], preferred_element_type=jnp.float32)
         + jnp.dot(mi, w[d_in:2 * d_in], preferred_element_type=jnp.float32)
         + jnp.dot(mo, w[2 * d_in:3 * d_in], preferred_element_type=jnp.float32))
    o_ref[...] = jnp.maximum(o + b_ref[...], 0.0)

  nb = n // blk
  return pl.pallas_call(
      body,
      grid=grid,
      in_specs=[
          pl.BlockSpec((blk, d_in), lambda i: (i, 0)),
          pl.BlockSpec((1, blk, d_in), lambda i: (0, i, 0)),
          pl.BlockSpec((1, blk, d_in), lambda i: (1, i, 0)),
          pl.BlockSpec((blk, 1), lambda i: (i, 0)),
          pl.BlockSpec((blk, 1), lambda i: (nb + i, 0)),
          pl.BlockSpec((3 * d_in, d_out), lambda i: (0, 0)),
          pl.BlockSpec((1, d_out), lambda i: (0, 0)),
      ],
      out_specs=pl.BlockSpec((blk, d_out), lambda i: (i, 0)),
      out_shape=jax.ShapeDtypeStruct((n, d_out), jnp.float32),
  )(x, acc, acc, deg, deg, wt, b2)


def kernel(x, edge_index, W, b):
  n, d_in = x.shape
  d_out = W.shape[0]
  e = edge_index.shape[1]

  src = edge_index[0].astype(jnp.int32)
  dst = edge_index[1].astype(jnp.int32)

  chunks = -(-e // (NS * CHUNK))
  per_tile_chunks = -(-chunks // IDX_BLK) * IDX_BLK
  e_pad = per_tile_chunks * NS * CHUNK
  pad = e_pad - e
  g0 = jnp.pad(src, (0, pad), constant_values=n)
  g1 = jnp.pad(dst, (0, pad), constant_values=n)
  idx2 = jnp.stack([g0, g1]).reshape(NC, NS, per_tile_chunks, CHUNK)

  n_rows = -(-(n + 1) // (NS * 128)) * NS * 128
  deg_rows = n_rows // 128

  zeros_sums = jnp.zeros((n_rows // NS, d_in), jnp.bfloat16)
  zeros_deg = jnp.zeros((deg_rows, 128), jnp.float32)

  x_bf = x.astype(jnp.bfloat16)
  acc, deg = _sc_sums(x_bf, idx2, zeros_sums, zeros_deg, n_rows,
                      per_tile_chunks, d_in)
  deg2 = deg.reshape(NC, n_rows)[:, :n].reshape(NC * n, 1)

  wt = W.T
  b2 = b.reshape(1, d_out)
  return _tc_combine(x, acc, deg2, wt, b2, d_in, d_out)


# speedup vs baseline: 4.1772x; 1.0269x over previous
import functools

import jax
import jax.numpy as jnp
from jax import lax
from jax.experimental import pallas as pl
from jax.experimental.pallas import tpu as pltpu
from jax.experimental.pallas import tpu_sc as plsc

NC = 2
NS = 16
CHUNK = 128
LANES = 16
IDX_BLK = 16


def _sc_sums(x, idx2, zeros_init, zeros_deg, n_rows, n_chunks, d):
  rows_per_tile = n_rows // NS
  deg_rows = n_rows // 128
  mesh = plsc.VectorSubcoreMesh(
      core_axis_name="c", subcore_axis_name="s", num_cores=NC, num_subcores=NS)

  @functools.partial(
      pl.kernel,
      out_type=(jax.ShapeDtypeStruct((NC, n_rows, d), jnp.bfloat16),
                jax.ShapeDtypeStruct((NC, deg_rows, 128), jnp.float32)),
      mesh=mesh,
      compiler_params=pltpu.CompilerParams(use_tc_tiling_on_sc=False,
                                           needs_layout_passes=False),
      scratch_types=[
          pltpu.VMEM((3 * IDX_BLK, CHUNK), jnp.int32),
          pltpu.VMEM((3 * IDX_BLK, CHUNK), jnp.int32),
          pltpu.VMEM((3, CHUNK, d), jnp.bfloat16),
          pltpu.VMEM((deg_rows, 128), jnp.float32),
          pltpu.VMEM((deg_rows,), jnp.int32),
          pltpu.VMEM_SHARED((x.shape[0] + 16, d), jnp.bfloat16),
          pltpu.VMEM_SHARED((n_rows, d), jnp.bfloat16),
          pltpu.VMEM_SHARED((deg_rows, 128), jnp.float32),
          pltpu.SemaphoreType.DMA,
          pltpu.SemaphoreType.DMA,
          pltpu.SemaphoreType.DMA,
      ],
  )
  def agg(x_hbm, idx_hbm, z_hbm, zd_hbm, out_hbm, deg_hbm,
          g_v, s_v, rows_v, deg_h, deg_i, x_sh, acc, deg_acc,
          gsem, ssem, isem):
    c = lax.axis_index("c")
    s = lax.axis_index("s")
    r0 = s * rows_per_tile
    n_blocks = n_chunks // IDX_BLK
    n_x = x_hbm.shape[0]
    xs0 = s * (n_x // NS)
    pltpu.sync_copy(x_hbm.at[pl.ds(xs0, n_x // NS)],
                    x_sh.at[pl.ds(xs0, n_x // NS)])
    pltpu.sync_copy(z_hbm, acc.at[pl.ds(r0, rows_per_tile), :])
    pltpu.sync_copy(zd_hbm, deg_h)

    @pl.when(s == 0)
    def _():
      pltpu.sync_copy(zd_hbm, deg_acc)

    for k in range(deg_rows // LANES):
      deg_i[pl.ds(k * LANES, LANES)] = lax.iota(jnp.int32, LANES) + k * LANES

    def stage_idx(bi):
      slot = (bi % 3) * IDX_BLK
      src = pl.ds(bi * IDX_BLK, IDX_BLK)
      dstv = pl.ds(slot, IDX_BLK)
      pltpu.async_copy(idx_hbm.at[c, s, src], g_v.at[dstv], isem)
      pltpu.async_copy(idx_hbm.at[1 - c, s, src], s_v.at[dstv], isem)

    def wait_idx():
      pltpu.make_async_copy(idx_hbm.at[0, 0, pl.ds(0, IDX_BLK)],
                            g_v.at[pl.ds(0, IDX_BLK)], isem).wait()
      pltpu.make_async_copy(idx_hbm.at[0, 0, pl.ds(0, IDX_BLK)],
                            s_v.at[pl.ds(0, IDX_BLK)], isem).wait()

    def start_gather(j):
      pltpu.async_copy(x_sh.at[g_v.at[j % (3 * IDX_BLK)]],
                       rows_v.at[j % 3], gsem)

    def wait_gather():
      pltpu.make_async_copy(x_sh.at[g_v.at[0]], rows_v.at[0], gsem).wait()

    def start_scatter(j):
      pltpu.async_copy(rows_v.at[j % 3], acc.at[s_v.at[j % (3 * IDX_BLK)]],
                       ssem, add=True)

    def wait_scatter():
      pltpu.make_async_copy(rows_v.at[0], acc.at[s_v.at[0]], ssem).wait()

    stage_idx(0)
    wait_idx()
    plsc.subcore_barrier()
    start_gather(0)
    ones = jnp.ones((LANES,), jnp.float32)

    def body(j, carry):
      jmod = j % (3 * IDX_BLK)
      wait_gather()

      @pl.when(j > 1)
      def _():
        wait_scatter()

      @pl.when(jnp.logical_and(j % IDX_BLK == 0, j // IDX_BLK + 1 < n_blocks))
      def _():
        stage_idx(j // IDX_BLK + 1)

      @pl.when(j + 1 < n_chunks)
      def _():
        @pl.when((j + 1) % IDX_BLK == 0)
        def _():
          wait_idx()
        start_gather(j + 1)

      start_scatter(j)
      for k in range(CHUNK // LANES):
        v = s_v[jmod, pl.ds(k * LANES, LANES)]
        hi = lax.shift_right_logical(v, 7)
        lo = lax.bitwise_and(v, 127)
        plsc.addupdate_scatter(deg_h, [hi, lo], ones)
      return carry

    lax.fori_loop(0, n_chunks, body, 0)
    wait_scatter()
    wait_scatter()
    pltpu.sync_copy(deg_h, deg_acc.at[deg_i], add=True)
    plsc.subcore_barrier()
    pltpu.sync_copy(acc.at[pl.ds(r0, rows_per_tile), :],
                    out_hbm.at[c, pl.ds(r0, rows_per_tile), :])

    @pl.when(s == 0)
    def _():
      pltpu.sync_copy(deg_acc, deg_hbm.at[c])

  return agg(x, idx2, zeros_init, zeros_deg)


def _tc_combine(x, acc, deg, wt, b2, d_in, d_out):
  n = x.shape[0]
  blk = 1000
  grid = (n // blk,)

  def body(x_ref, ai_ref, ao_ref, di_ref, do_ref, w_ref, b_ref, o_ref):
    xb = x_ref[...]
    mi = ai_ref[0].astype(jnp.float32) / jnp.maximum(di_ref[...], 1.0)
    mo = ao_ref[0].astype(jnp.float32) / jnp.maximum(do_ref[...], 1.0)
    w = w_ref[...]
    o = (jnp.dot(xb, w[:d_in], preferred_element_type=jnp.float32)
         + jnp.dot(mi, w[d_in:2 * d_in], preferred_element_type=jnp.float32)
         + jnp.dot(mo, w[2 * d_in:3 * d_in], preferred_element_type=jnp.float32))
    o_ref[...] = jnp.maximum(o + b_ref[...], 0.0)

  nb = n // blk
  return pl.pallas_call(
      body,
      grid=grid,
      in_specs=[
          pl.BlockSpec((blk, d_in), lambda i: (i, 0)),
          pl.BlockSpec((1, blk, d_in), lambda i: (0, i, 0)),
          pl.BlockSpec((1, blk, d_in), lambda i: (1, i, 0)),
          pl.BlockSpec((blk, 1), lambda i: (i, 0)),
          pl.BlockSpec((blk, 1), lambda i: (nb + i, 0)),
          pl.BlockSpec((3 * d_in, d_out), lambda i: (0, 0)),
          pl.BlockSpec((1, d_out), lambda i: (0, 0)),
      ],
      out_specs=pl.BlockSpec((blk, d_out), lambda i: (i, 0)),
      out_shape=jax.ShapeDtypeStruct((n, d_out), jnp.float32),
  )(x, acc, acc, deg, deg, wt, b2)


def kernel(x, edge_index, W, b):
  n, d_in = x.shape
  d_out = W.shape[0]
  e = edge_index.shape[1]

  src = edge_index[0].astype(jnp.int32)
  dst = edge_index[1].astype(jnp.int32)

  chunks = -(-e // (NS * CHUNK))
  per_tile_chunks = -(-chunks // IDX_BLK) * IDX_BLK
  e_pad = per_tile_chunks * NS * CHUNK
  pad = e_pad - e
  g0 = jnp.pad(src, (0, pad), constant_values=n)
  g1 = jnp.pad(dst, (0, pad), constant_values=n)
  idx2 = jnp.stack([g0, g1]).reshape(NC, NS, per_tile_chunks, CHUNK)

  n_rows = -(-(n + 1) // (NS * 128)) * NS * 128
  deg_rows = n_rows // 128

  zeros_sums = jnp.zeros((n_rows // NS, d_in), jnp.bfloat16)
  zeros_deg = jnp.zeros((deg_rows, 128), jnp.float32)

  x_bf = x.astype(jnp.bfloat16)
  acc, deg = _sc_sums(x_bf, idx2, zeros_sums, zeros_deg, n_rows,
                      per_tile_chunks, d_in)
  deg2 = deg.reshape(NC, n_rows)[:, :n].reshape(NC * n, 1)

  wt = W.T
  b2 = b.reshape(1, d_out)
  return _tc_combine(x, acc, deg2, wt, b2, d_in, d_out)


# speedup vs baseline: 4.1890x; 1.0028x over previous
import functools

import jax
import jax.numpy as jnp
from jax import lax
from jax.experimental import pallas as pl
from jax.experimental.pallas import tpu as pltpu
from jax.experimental.pallas import tpu_sc as plsc

NC = 2
NS = 16
CHUNK = 128
LANES = 16
IDX_BLK = 16


def _sc_sums(x, idx2, zeros_init, zeros_deg, n_rows, n_chunks, d):
  rows_per_tile = n_rows // NS
  deg_rows = n_rows // 128
  mesh = plsc.VectorSubcoreMesh(
      core_axis_name="c", subcore_axis_name="s", num_cores=NC, num_subcores=NS)

  @functools.partial(
      pl.kernel,
      out_type=(jax.ShapeDtypeStruct((NC, n_rows, d), jnp.bfloat16),
                jax.ShapeDtypeStruct((NC, deg_rows, 128), jnp.float32)),
      mesh=mesh,
      compiler_params=pltpu.CompilerParams(use_tc_tiling_on_sc=False,
                                           needs_layout_passes=False),
      scratch_types=[
          pltpu.VMEM((3 * IDX_BLK, CHUNK), jnp.int32),
          pltpu.VMEM((3 * IDX_BLK, CHUNK), jnp.int32),
          pltpu.VMEM((3, CHUNK, d), jnp.bfloat16),
          pltpu.VMEM((deg_rows, 128), jnp.float32),
          pltpu.VMEM((deg_rows,), jnp.int32),
          pltpu.VMEM_SHARED((x.shape[0] + 16, d), jnp.bfloat16),
          pltpu.VMEM_SHARED((n_rows, d), jnp.bfloat16),
          pltpu.VMEM_SHARED((deg_rows, 128), jnp.float32),
          pltpu.SemaphoreType.DMA,
          pltpu.SemaphoreType.DMA,
          pltpu.SemaphoreType.DMA,
      ],
  )
  def agg(x_hbm, idx_hbm, z_hbm, zd_hbm, out_hbm, deg_hbm,
          g_v, s_v, rows_v, deg_h, deg_i, x_sh, acc, deg_acc,
          gsem, ssem, isem):
    c = lax.axis_index("c")
    s = lax.axis_index("s")
    r0 = s * rows_per_tile
    n_blocks = n_chunks // IDX_BLK
    n_x = x_hbm.shape[0]
    xs0 = s * (n_x // NS)
    pltpu.sync_copy(x_hbm.at[pl.ds(xs0, n_x // NS)],
                    x_sh.at[pl.ds(xs0, n_x // NS)])
    pltpu.sync_copy(z_hbm, acc.at[pl.ds(r0, rows_per_tile), :])
    pltpu.sync_copy(zd_hbm, deg_h)

    @pl.when(s == 0)
    def _():
      pltpu.sync_copy(zd_hbm, deg_acc)

    for k in range(deg_rows // LANES):
      deg_i[pl.ds(k * LANES, LANES)] = lax.iota(jnp.int32, LANES) + k * LANES

    def stage_idx(bi):
      slot = (bi % 3) * IDX_BLK
      src = pl.ds(bi * IDX_BLK, IDX_BLK)
      dstv = pl.ds(slot, IDX_BLK)
      pltpu.async_copy(idx_hbm.at[c, s, src], g_v.at[dstv], isem)
      pltpu.async_copy(idx_hbm.at[1 - c, s, src], s_v.at[dstv], isem)

    def wait_idx():
      pltpu.make_async_copy(idx_hbm.at[0, 0, pl.ds(0, IDX_BLK)],
                            g_v.at[pl.ds(0, IDX_BLK)], isem).wait()
      pltpu.make_async_copy(idx_hbm.at[0, 0, pl.ds(0, IDX_BLK)],
                            s_v.at[pl.ds(0, IDX_BLK)], isem).wait()

    def start_gather(j):
      pltpu.async_copy(x_sh.at[g_v.at[j % (3 * IDX_BLK)]],
                       rows_v.at[j % 3], gsem)

    def wait_gather():
      pltpu.make_async_copy(x_sh.at[g_v.at[0]], rows_v.at[0], gsem).wait()

    def start_scatter(j):
      pltpu.async_copy(rows_v.at[j % 3], acc.at[s_v.at[j % (3 * IDX_BLK)]],
                       ssem, add=True)

    def wait_scatter():
      pltpu.make_async_copy(rows_v.at[0], acc.at[s_v.at[0]], ssem).wait()

    stage_idx(0)
    wait_idx()
    plsc.subcore_barrier()
    start_gather(0)
    ones = jnp.ones((LANES,), jnp.float32)

    def body(j, carry):
      jmod = j % (3 * IDX_BLK)
      wait_gather()

      @pl.when(j > 1)
      def _():
        wait_scatter()

      @pl.when(jnp.logical_and(j % IDX_BLK == 0, j // IDX_BLK + 1 < n_blocks))
      def _():
        stage_idx(j // IDX_BLK + 1)

      @pl.when(j + 1 < n_chunks)
      def _():
        @pl.when((j + 1) % IDX_BLK == 0)
        def _():
          wait_idx()
        start_gather(j + 1)

      start_scatter(j)
      for k in range(CHUNK // LANES):
        v = s_v[jmod, pl.ds(k * LANES, LANES)]
        hi = lax.shift_right_logical(v, 7)
        lo = lax.bitwise_and(v, 127)
        plsc.addupdate_scatter(deg_h, [hi, lo], ones)
      return carry

    lax.fori_loop(0, n_chunks, body, 0)
    wait_scatter()
    wait_scatter()
    pltpu.sync_copy(deg_h, deg_acc.at[deg_i], add=True)
    plsc.subcore_barrier()
    pltpu.sync_copy(acc.at[pl.ds(r0, rows_per_tile), :],
                    out_hbm.at[c, pl.ds(r0, rows_per_tile), :])

    @pl.when(s == 0)
    def _():
      pltpu.sync_copy(deg_acc, deg_hbm.at[c])

  return agg(x, idx2, zeros_init, zeros_deg)


def _tc_combine(x, acc, deg, wt, b2, d_in, d_out):
  n = x.shape[0]
  blk = 1000
  grid = (n // blk,)

  def body(x_ref, ai_ref, ao_ref, di_ref, do_ref, w_ref, b_ref, o_ref):
    xb = x_ref[...]
    ri = 1.0 / jnp.maximum(di_ref[...], 1.0)
    ro = 1.0 / jnp.maximum(do_ref[...], 1.0)
    w = w_ref[...]
    o = (jnp.dot(xb, w[:d_in], preferred_element_type=jnp.float32)
         + ri * jnp.dot(ai_ref[0], w[d_in:2 * d_in],
                        preferred_element_type=jnp.float32)
         + ro * jnp.dot(ao_ref[0], w[2 * d_in:3 * d_in],
                        preferred_element_type=jnp.float32))
    o_ref[...] = jnp.maximum(o + b_ref[...], 0.0)

  nb = n // blk
  return pl.pallas_call(
      body,
      grid=grid,
      in_specs=[
          pl.BlockSpec((blk, d_in), lambda i: (i, 0)),
          pl.BlockSpec((1, blk, d_in), lambda i: (0, i, 0)),
          pl.BlockSpec((1, blk, d_in), lambda i: (1, i, 0)),
          pl.BlockSpec((blk, 1), lambda i: (i, 0)),
          pl.BlockSpec((blk, 1), lambda i: (nb + i, 0)),
          pl.BlockSpec((3 * d_in, d_out), lambda i: (0, 0)),
          pl.BlockSpec((1, d_out), lambda i: (0, 0)),
      ],
      out_specs=pl.BlockSpec((blk, d_out), lambda i: (i, 0)),
      out_shape=jax.ShapeDtypeStruct((n, d_out), jnp.float32),
  )(x, acc, acc, deg, deg, wt, b2)


def kernel(x, edge_index, W, b):
  n, d_in = x.shape
  d_out = W.shape[0]
  e = edge_index.shape[1]

  src = edge_index[0].astype(jnp.int32)
  dst = edge_index[1].astype(jnp.int32)

  chunks = -(-e // (NS * CHUNK))
  per_tile_chunks = -(-chunks // IDX_BLK) * IDX_BLK
  e_pad = per_tile_chunks * NS * CHUNK
  pad = e_pad - e
  g0 = jnp.pad(src, (0, pad), constant_values=n)
  g1 = jnp.pad(dst, (0, pad), constant_values=n)
  idx2 = jnp.stack([g0, g1]).reshape(NC, NS, per_tile_chunks, CHUNK)

  n_rows = -(-(n + 1) // (NS * 128)) * NS * 128
  deg_rows = n_rows // 128

  zeros_sums = jnp.zeros((n_rows // NS, d_in), jnp.bfloat16)
  zeros_deg = jnp.zeros((deg_rows, 128), jnp.float32)

  x_bf = x.astype(jnp.bfloat16)
  acc, deg = _sc_sums(x_bf, idx2, zeros_sums, zeros_deg, n_rows,
                      per_tile_chunks, d_in)
  deg2 = deg.reshape(NC, n_rows)[:, :n].reshape(NC * n, 1)

  wt = W.T.astype(jnp.bfloat16)
  b2 = b.reshape(1, d_out)
  return _tc_combine(x_bf, acc, deg2, wt, b2, d_in, d_out)
